# Initial kernel scaffold; baseline (speedup 1.0000x reference)
import jax
import jax.numpy as jnp
from jax.experimental import pallas as pl


def kernel(x, adj, W1, b1, W2, b2, W3, b3):
    raise NotImplementedError("write your pallas kernel here")


# speedup vs baseline: 3.8792x; 3.8792x over previous
import jax
import jax.numpy as jnp
from jax import lax
from jax.experimental import pallas as pl
from jax.experimental.pallas import tpu as pltpu
from jax.experimental.pallas import tpu_sc as plsc

N = 10000
E = 320000
D_IN = 128
D_HID = 256
D_OUT = 128

C = 128
IB = 16
E_PAD = 327680
NCHUNK = E_PAD // C
CPW_ES = NCHUNK // 32
CPW_CS = NCHUNK // 16
ACC_ROWS = 10112
ZROWS = ACC_ROWS // 16
OROWS = ACC_ROWS // 16

_MESH = plsc.VectorSubcoreMesh(
    core_axis_name="c", subcore_axis_name="s", num_cores=2, num_subcores=16)


def _agg_pass(y_hbm, srcs, dsts, acc, src_v, dst_v, rows0, rows1,
              sem0, sem1, base_chunk, nblocks):
  def outer(g, carry):
    blk = pl.multiple_of(base_chunk + g * IB, IB)
    pltpu.sync_copy(srcs.at[pl.ds(blk, IB)], src_v)
    pltpu.sync_copy(dsts.at[pl.ds(blk, IB)], dst_v)
    pltpu.async_copy(y_hbm.at[src_v.at[0]], rows0, sem0)

    def inner(p, carry2):
      j0 = 2 * p
      pltpu.async_copy(y_hbm.at[src_v.at[j0 + 1]], rows1, sem1)
      pltpu.make_async_copy(y_hbm.at[src_v.at[j0]], rows0, sem0).wait()
      pltpu.sync_copy(rows0, acc.at[dst_v.at[j0]], add=True)

      @pl.when(p < IB // 2 - 1)
      def _():
        pltpu.async_copy(y_hbm.at[src_v.at[j0 + 2]], rows0, sem0)

      pltpu.make_async_copy(y_hbm.at[src_v.at[j0]], rows1, sem1).wait()
      pltpu.sync_copy(rows1, acc.at[dst_v.at[j0 + 1]], add=True)
      return carry2

    lax.fori_loop(0, IB // 2, inner, 0)
    return carry

  lax.fori_loop(0, nblocks, outer, 0)


def _zero_acc(zinit, acc, s):
  pltpu.sync_copy(zinit.at[pl.ds(s * ZROWS, ZROWS)],
                  acc.at[pl.ds(s * ZROWS, ZROWS)])


def _writeout(acc, o, s):
  pltpu.sync_copy(acc.at[pl.ds(s * OROWS, OROWS)],
                  o.at[pl.ds(s * OROWS, OROWS)])


def _agg_edge_split_body(x_hbm, srcs, dsts, zinit, q0, q1,
                         src_v, dst_v, rows0, rows1, acc, sem0, sem1):
  c = lax.axis_index("c")
  s = lax.axis_index("s")
  w = c * 16 + s
  _zero_acc(zinit, acc, s)
  plsc.subcore_barrier()
  _agg_pass(x_hbm, srcs, dsts, acc, src_v, dst_v, rows0, rows1, sem0, sem1,
            w * CPW_ES, CPW_ES // IB)
  plsc.subcore_barrier()

  @pl.when(c == 0)
  def _():
    _writeout(acc, q0, s)

  @pl.when(c == 1)
  def _():
    _writeout(acc, q1, s)


def _agg_col_split_body(ya, yb, srcs, dsts, zinit, a0, a1,
                        src_v, dst_v, rows0, rows1, acc, sem0, sem1):
  c = lax.axis_index("c")
  s = lax.axis_index("s")
  _zero_acc(zinit, acc, s)
  plsc.subcore_barrier()

  @pl.when(c == 0)
  def _():
    _agg_pass(ya, srcs, dsts, acc, src_v, dst_v, rows0, rows1, sem0, sem1,
              s * CPW_CS, CPW_CS // IB)

  @pl.when(c == 1)
  def _():
    _agg_pass(yb, srcs, dsts, acc, src_v, dst_v, rows0, rows1, sem0, sem1,
              s * CPW_CS, CPW_CS // IB)

  plsc.subcore_barrier()

  @pl.when(c == 0)
  def _():
    _writeout(acc, a0, s)

  @pl.when(c == 1)
  def _():
    _writeout(acc, a1, s)


def _make_agg(body):
  return pl.kernel(
      body,
      out_type=(jax.ShapeDtypeStruct((ACC_ROWS, 128), jnp.float32),
                jax.ShapeDtypeStruct((ACC_ROWS, 128), jnp.float32)),
      mesh=_MESH,
      scratch_types=(
          pltpu.VMEM((IB, C), jnp.int32),
          pltpu.VMEM((IB, C), jnp.int32),
          pltpu.VMEM((C, 128), jnp.float32),
          pltpu.VMEM((C, 128), jnp.float32),
          pltpu.VMEM_SHARED((ACC_ROWS, 128), jnp.float32),
          pltpu.SemaphoreType.DMA,
          pltpu.SemaphoreType.DMA,
      ),
  )


_agg_edge_split = _make_agg(_agg_edge_split_body)
_agg_col_split = _make_agg(_agg_col_split_body)

RB = 1000


def _mm1_body(p0, p1, w1, b1, ya, yb):
  h = p0[...] + p1[...]
  y = jnp.dot(h, w1[...], preferred_element_type=jnp.float32) + b1[...]
  y = jnp.maximum(y, 0.0)
  ya[...] = y[:, :128]
  yb[...] = y[:, 128:]


_mm1 = pl.pallas_call(
    _mm1_body,
    grid=(N // RB,),
    in_specs=[
        pl.BlockSpec((RB, D_IN), lambda i: (i, 0)),
        pl.BlockSpec((RB, D_IN), lambda i: (i, 0)),
        pl.BlockSpec((D_IN, D_HID), lambda i: (0, 0)),
        pl.BlockSpec((1, D_HID), lambda i: (0, 0)),
    ],
    out_specs=[
        pl.BlockSpec((RB, 128), lambda i: (i, 0)),
        pl.BlockSpec((RB, 128), lambda i: (i, 0)),
    ],
    out_shape=[
        jax.ShapeDtypeStruct((N, 128), jnp.float32),
        jax.ShapeDtypeStruct((N, 128), jnp.float32),
    ],
)


def _mm2_body(a0, a1, w2a, w2b, b2, w3, t3):
  h = (jnp.dot(a0[...], w2a[...], preferred_element_type=jnp.float32)
       + jnp.dot(a1[...], w2b[...], preferred_element_type=jnp.float32)
       + b2[...])
  h = jnp.maximum(h, 0.0)
  t3[...] = jnp.dot(h, w3[...], preferred_element_type=jnp.float32)


_mm2 = pl.pallas_call(
    _mm2_body,
    grid=(N // RB,),
    in_specs=[
        pl.BlockSpec((RB, 128), lambda i: (i, 0)),
        pl.BlockSpec((RB, 128), lambda i: (i, 0)),
        pl.BlockSpec((128, D_HID), lambda i: (0, 0)),
        pl.BlockSpec((128, D_HID), lambda i: (0, 0)),
        pl.BlockSpec((1, D_HID), lambda i: (0, 0)),
        pl.BlockSpec((D_HID, D_OUT), lambda i: (0, 0)),
    ],
    out_specs=pl.BlockSpec((RB, D_OUT), lambda i: (i, 0)),
    out_shape=jax.ShapeDtypeStruct((N, D_OUT), jnp.float32),
)


def _mm3_body(q0, q1, b3, out):
  out[...] = q0[...] + q1[...] + b3[...]


_mm3 = pl.pallas_call(
    _mm3_body,
    grid=(N // RB,),
    in_specs=[
        pl.BlockSpec((RB, D_OUT), lambda i: (i, 0)),
        pl.BlockSpec((RB, D_OUT), lambda i: (i, 0)),
        pl.BlockSpec((1, D_OUT), lambda i: (0, 0)),
    ],
    out_specs=pl.BlockSpec((RB, D_OUT), lambda i: (i, 0)),
    out_shape=jax.ShapeDtypeStruct((N, D_OUT), jnp.float32),
)


@jax.jit
def kernel(x, adj, W1, b1, W2, b2, W3, b3):
  src = adj[0].astype(jnp.int32)
  dst = adj[1].astype(jnp.int32)
  srcs = jnp.concatenate(
      [src, jnp.zeros((E_PAD - E,), jnp.int32)]).reshape(NCHUNK, C)
  dsts = jnp.concatenate(
      [dst, jnp.full((E_PAD - E,), N, jnp.int32)]).reshape(NCHUNK, C)
  zinit = jnp.zeros((ACC_ROWS, 128), jnp.float32)

  q0, q1 = _agg_edge_split(x, srcs, dsts, zinit)
  ya, yb = _mm1(q0, q1, W1, b1.reshape(1, D_HID))
  a0, a1 = _agg_col_split(ya, yb, srcs, dsts, zinit)
  t3 = _mm2(a0, a1, W2[:128], W2[128:], b2.reshape(1, D_HID), W3)
  r0, r1 = _agg_edge_split(t3, srcs, dsts, zinit)
  return _mm3(r0, r1, b3.reshape(1, D_OUT))
